# baseline (speedup 1.0000x reference)
import functools

import jax
import jax.numpy as jnp
from jax import lax
from jax.experimental import pallas as pl
from jax.experimental.pallas import tpu as pltpu
from jax.experimental.pallas import tpu_sc as plsc

_NUM_CLASSES = 100
_BATCH = 4096
_LAMBDA = 3.0
_NUM_WORKERS = 32
_F = 2048
_NTC = _BATCH - _F
_B_PER_W = _F // _NUM_WORKERS
_NSLAB = 8
_TC_GROUP = 8
_TC_STEPS = _NTC // _TC_GROUP


def _sc_gather(hist_t, idx):
    mesh = plsc.VectorSubcoreMesh(core_axis_name="c", subcore_axis_name="s")

    @functools.partial(
        pl.kernel,
        out_type=jax.ShapeDtypeStruct((_F, _NUM_CLASSES), jnp.float32),
        mesh=mesh,
        scratch_types=[
            pltpu.VMEM((_B_PER_W,), jnp.int32),
            [pltpu.VMEM((_NUM_CLASSES, 128), jnp.float32)] * _NSLAB,
            pltpu.VMEM((_B_PER_W, _NUM_CLASSES), jnp.float32),
            pltpu.SemaphoreType.DMA,
            pltpu.SemaphoreType.DMA,
        ],
        compiler_params=pltpu.CompilerParams(needs_layout_passes=False),
    )
    def gather_kernel(
        hist_hbm, idx_hbm, out_hbm, idx_v, slabs_v, compact_v, sem_a, sem_b
    ):
        wid = lax.axis_index("s") * 2 + lax.axis_index("c")
        base = wid * _B_PER_W
        pltpu.sync_copy(idx_hbm.at[pl.ds(base, _B_PER_W)], idx_v)

        iota16 = lax.iota(jnp.int32, 16)
        chunk_starts = [0, 16, 32, 48, 64, 80, 84]
        sems = [sem_a, sem_b]

        def issue(v, g, sem):
            ts = []
            for u in range(4):
                r = v[4 * g + u]
                t = lax.bitwise_and(r, 127)
                off = pl.multiple_of(lax.sub(r, t), 128)
                ts.append(t)
                pltpu.async_copy(
                    hist_hbm.at[:, pl.ds(off, 128)],
                    slabs_v[4 * (g % 2) + u],
                    sem,
                )
            return ts

        def extract(k, g, ts):
            for u in range(4):
                row = k * 16 + 4 * g + u
                tvec = iota16 * 0 + ts[u]
                for c0 in chunk_starts:
                    vals = plsc.load_gather(
                        slabs_v[4 * (g % 2) + u], [c0 + iota16, tvec]
                    )
                    compact_v[row, pl.ds(c0, 16)] = vals

        def drain(sem):
            for u in range(4):
                pltpu.make_async_copy(
                    hist_hbm.at[:, pl.ds(0, 128)], slabs_v[u], sem
                ).wait()

        def body(k, _):
            v = idx_v[pl.ds(k * 16, 16)]
            ts0 = issue(v, 0, sems[0])
            ts1 = issue(v, 1, sems[1])
            drain(sems[0])
            extract(k, 0, ts0)
            ts2 = issue(v, 2, sems[0])
            drain(sems[1])
            extract(k, 1, ts1)
            ts3 = issue(v, 3, sems[1])
            drain(sems[0])
            extract(k, 2, ts2)
            drain(sems[1])
            extract(k, 3, ts3)
            return 0

        lax.fori_loop(0, _B_PER_W // 16, body, 0)
        pltpu.sync_copy(compact_v, out_hbm.at[pl.ds(base, _B_PER_W)])

    return gather_kernel(hist_t, idx)


def _tc_gather_body(off_ref, t_ref, hist_ref, out_ref, slabs_ref, sems):
    g = pl.program_id(0)

    def issue(gg, b):
        for u in range(_TC_GROUP):
            off = pl.multiple_of(off_ref[gg * _TC_GROUP + u], 128)
            pltpu.make_async_copy(
                hist_ref.at[:, pl.ds(off, 128)],
                slabs_ref.at[b, :, pl.ds(128 * u, 128)],
                sems.at[b],
            ).start()

    @pl.when(g == 0)
    def _prologue():
        issue(0, 0)

    @pl.when(g + 1 < _TC_STEPS)
    def _next():
        issue(g + 1, (g + 1) % 2)

    b = g % 2
    for u in range(_TC_GROUP):
        pltpu.make_async_copy(
            hist_ref.at[:, pl.ds(0, 128)],
            slabs_ref.at[b, :, pl.ds(128 * u, 128)],
            sems.at[b],
        ).wait()

    s = slabs_ref[b]
    tcol = jnp.reshape(t_ref[0, 0, :], (_TC_GROUP, 1))
    li = lax.broadcasted_iota(jnp.int32, (_TC_GROUP, 128 * _TC_GROUP), 1)
    si = lax.broadcasted_iota(jnp.int32, (_TC_GROUP, 128 * _TC_GROUP), 0)
    p8 = jnp.where(
        (lax.shift_right_logical(li, 7) == si)
        & (lax.bitwise_and(li, 127) == tcol),
        1.0,
        0.0,
    ).astype(jnp.float32)
    e = lax.dot_general(
        s, p8, (((1,), (1,)), ((), ())), preferred_element_type=jnp.float32
    )
    m = g % 16
    ri = lax.broadcasted_iota(jnp.int32, (_TC_GROUP, 128), 0)
    rl = lax.broadcasted_iota(jnp.int32, (_TC_GROUP, 128), 1)
    rmat = jnp.where(rl == m * _TC_GROUP + ri, 1.0, 0.0).astype(jnp.float32)
    z = lax.dot_general(
        e, rmat, (((1,), (0,)), ((), ())), preferred_element_type=jnp.float32
    )

    @pl.when(m == 0)
    def _init():
        out_ref[...] = z

    @pl.when(m != 0)
    def _accum():
        out_ref[...] += z


def _tc_gather(hist_t, off_tc, t_tc):
    grid_spec = pltpu.PrefetchScalarGridSpec(
        num_scalar_prefetch=0,
        grid=(_TC_STEPS,),
        in_specs=[
            pl.BlockSpec(memory_space=pltpu.SMEM),
            pl.BlockSpec((1, 1, _TC_GROUP), lambda g: (g, 0, 0)),
            pl.BlockSpec(memory_space=pl.ANY),
        ],
        out_specs=pl.BlockSpec(
            (_NUM_CLASSES, 128), lambda g: (0, g // 16)
        ),
        scratch_shapes=[
            pltpu.VMEM((2, _NUM_CLASSES, 128 * _TC_GROUP), jnp.float32),
            pltpu.SemaphoreType.DMA((2,)),
        ],
    )
    return pl.pallas_call(
        _tc_gather_body,
        grid_spec=grid_spec,
        out_shape=jax.ShapeDtypeStruct((_NUM_CLASSES, _NTC), jnp.float32),
    )(off_tc, t_tc, hist_t)


def _tc_softmax_body(out_ref, tgt_ref, y_ref, ce_ref):
    x = out_ref[...]
    m = jnp.max(x, axis=0, keepdims=True)
    xm = x - m
    e = jnp.exp(xm)
    s = jnp.sum(e, axis=0, keepdims=True)
    y_ref[...] = jnp.clip(e / s, 0.0001, 1.0 - 0.0001)
    log_sm = xm - jnp.log(s)
    ce_ref[0, 0] = jnp.sum(-tgt_ref[...] * log_sm)


def _tc_softmax(out_t, tgt_t):
    return pl.pallas_call(
        _tc_softmax_body,
        out_shape=(
            jax.ShapeDtypeStruct((_NUM_CLASSES, _BATCH), jnp.float32),
            jax.ShapeDtypeStruct((1, 1), jnp.float32),
        ),
        in_specs=[
            pl.BlockSpec(memory_space=pltpu.VMEM),
            pl.BlockSpec(memory_space=pltpu.VMEM),
        ],
        out_specs=(
            pl.BlockSpec(memory_space=pltpu.VMEM),
            pl.BlockSpec(memory_space=pltpu.SMEM),
        ),
    )(out_t, tgt_t)


def _tc_final_body(hsc_ref, htc_ref, y_ref, ce_ref, loss_ref):
    ii = lax.broadcasted_iota(jnp.int32, (_NUM_CLASSES, _NUM_CLASSES), 0)
    jj = lax.broadcasted_iota(jnp.int32, (_NUM_CLASSES, _NUM_CLASSES), 1)
    eye = jnp.where(ii == jj, 1.0, 0.0).astype(jnp.float32)
    hsc_t = lax.dot_general(
        eye,
        hsc_ref[...],
        (((1,), (1,)), ((), ())),
        preferred_element_type=jnp.float32,
    )
    y = y_ref[...]
    dot_a = jnp.sum(hsc_t * y[:, : _F], axis=0, keepdims=True)
    dot_b = jnp.sum(htc_ref[...] * y[:, _F:], axis=0, keepdims=True)
    reg = jnp.sum(jnp.log(1.0 - dot_a)) + jnp.sum(jnp.log(1.0 - dot_b))
    loss_ref[0, 0] = (ce_ref[0, 0] + _LAMBDA * reg) / _BATCH


def _tc_final(hist_sc, hist_tc_t, y_t, ce):
    return pl.pallas_call(
        _tc_final_body,
        out_shape=jax.ShapeDtypeStruct((1, 1), jnp.float32),
        in_specs=[
            pl.BlockSpec(memory_space=pltpu.VMEM),
            pl.BlockSpec(memory_space=pltpu.VMEM),
            pl.BlockSpec(memory_space=pltpu.VMEM),
            pl.BlockSpec(memory_space=pltpu.SMEM),
        ],
        out_specs=pl.BlockSpec(memory_space=pltpu.SMEM),
    )(hist_sc, hist_tc_t, y_t, ce)


def kernel(index, output, target, history):
    idx = index.astype(jnp.int32)
    t_all = jnp.bitwise_and(idx, 127)
    off_all = idx - t_all
    hist_t = history.T
    hist_sc = _sc_gather(hist_t, idx[:_F])
    hist_tc_t = _tc_gather(
        hist_t,
        off_all[_F:],
        jnp.reshape(t_all[_F:], (_TC_STEPS, 1, _TC_GROUP)),
    )
    y_t, ce = _tc_softmax(output.T, target.T)
    loss = _tc_final(hist_sc, hist_tc_t, y_t, ce)
    return loss[0, 0]


# speedup vs baseline: 2.1662x; 2.1662x over previous
import functools

import jax
import jax.numpy as jnp
from jax import lax
from jax.experimental import pallas as pl
from jax.experimental.pallas import tpu as pltpu
from jax.experimental.pallas import tpu_sc as plsc

_NUM_CLASSES = 100
_BATCH = 4096
_LAMBDA = 3.0
_NUM_WORKERS = 32
_F = 3584
_NTC = _BATCH - _F
_B_PER_W = _F // _NUM_WORKERS
_NSLAB = 8
_TC_GROUP = 8
_TC_STEPS = _NTC // _TC_GROUP


def _sc_gather(hist_t, idx):
    mesh = plsc.VectorSubcoreMesh(core_axis_name="c", subcore_axis_name="s")

    @functools.partial(
        pl.kernel,
        out_type=jax.ShapeDtypeStruct((_F, _NUM_CLASSES), jnp.float32),
        mesh=mesh,
        scratch_types=[
            pltpu.VMEM((_B_PER_W,), jnp.int32),
            [pltpu.VMEM((_NUM_CLASSES, 128), jnp.float32)] * _NSLAB,
            pltpu.VMEM((_B_PER_W, _NUM_CLASSES), jnp.float32),
            pltpu.SemaphoreType.DMA,
            pltpu.SemaphoreType.DMA,
        ],
        compiler_params=pltpu.CompilerParams(needs_layout_passes=False),
    )
    def gather_kernel(
        hist_hbm, idx_hbm, out_hbm, idx_v, slabs_v, compact_v, sem_a, sem_b
    ):
        wid = lax.axis_index("s") * 2 + lax.axis_index("c")
        base = wid * _B_PER_W
        pltpu.sync_copy(idx_hbm.at[pl.ds(base, _B_PER_W)], idx_v)

        iota16 = lax.iota(jnp.int32, 16)
        chunk_starts = [0, 16, 32, 48, 64, 80, 84]
        sems = [sem_a, sem_b]

        def issue(v, g, sem):
            ts = []
            for u in range(4):
                r = v[4 * g + u]
                t = lax.bitwise_and(r, 127)
                off = pl.multiple_of(lax.sub(r, t), 128)
                ts.append(t)
                pltpu.async_copy(
                    hist_hbm.at[:, pl.ds(off, 128)],
                    slabs_v[4 * (g % 2) + u],
                    sem,
                )
            return ts

        def extract(k, g, ts):
            for u in range(4):
                row = k * 16 + 4 * g + u
                tvec = iota16 * 0 + ts[u]
                for c0 in chunk_starts:
                    vals = plsc.load_gather(
                        slabs_v[4 * (g % 2) + u], [c0 + iota16, tvec]
                    )
                    compact_v[row, pl.ds(c0, 16)] = vals

        def drain(sem):
            for u in range(4):
                pltpu.make_async_copy(
                    hist_hbm.at[:, pl.ds(0, 128)], slabs_v[u], sem
                ).wait()

        def body(k, _):
            v = idx_v[pl.ds(k * 16, 16)]
            ts0 = issue(v, 0, sems[0])
            ts1 = issue(v, 1, sems[1])
            drain(sems[0])
            extract(k, 0, ts0)
            ts2 = issue(v, 2, sems[0])
            drain(sems[1])
            extract(k, 1, ts1)
            ts3 = issue(v, 3, sems[1])
            drain(sems[0])
            extract(k, 2, ts2)
            drain(sems[1])
            extract(k, 3, ts3)
            return 0

        lax.fori_loop(0, _B_PER_W // 16, body, 0)
        pltpu.sync_copy(compact_v, out_hbm.at[pl.ds(base, _B_PER_W)])

    return gather_kernel(hist_t, idx)


def _tc_gather_body(off_ref, t_ref, hist_ref, out_ref, slabs_ref, sems):
    g = pl.program_id(0)

    def issue(gg, b):
        for u in range(_TC_GROUP):
            off = pl.multiple_of(off_ref[gg * _TC_GROUP + u], 128)
            pltpu.make_async_copy(
                hist_ref.at[:, pl.ds(off, 128)],
                slabs_ref.at[b, :, pl.ds(128 * u, 128)],
                sems.at[b],
            ).start()

    @pl.when(g == 0)
    def _prologue():
        issue(0, 0)

    @pl.when(g + 1 < _TC_STEPS)
    def _next():
        issue(g + 1, (g + 1) % 2)

    b = g % 2
    pltpu.make_async_copy(
        hist_ref.at[:, pl.ds(0, 128 * _TC_GROUP)],
        slabs_ref.at[b],
        sems.at[b],
    ).wait()

    s = slabs_ref[b]
    tcol = jnp.reshape(t_ref[0, 0, :], (_TC_GROUP, 1))
    li = lax.broadcasted_iota(jnp.int32, (_TC_GROUP, 128 * _TC_GROUP), 1)
    si = lax.broadcasted_iota(jnp.int32, (_TC_GROUP, 128 * _TC_GROUP), 0)
    p8 = jnp.where(
        (lax.shift_right_logical(li, 7) == si)
        & (lax.bitwise_and(li, 127) == tcol),
        1.0,
        0.0,
    ).astype(jnp.float32)
    e = lax.dot_general(
        s, p8, (((1,), (1,)), ((), ())), preferred_element_type=jnp.float32
    )
    m = g % 16
    ri = lax.broadcasted_iota(jnp.int32, (_TC_GROUP, 128), 0)
    rl = lax.broadcasted_iota(jnp.int32, (_TC_GROUP, 128), 1)
    rmat = jnp.where(rl == m * _TC_GROUP + ri, 1.0, 0.0).astype(jnp.float32)
    z = lax.dot_general(
        e, rmat, (((1,), (0,)), ((), ())), preferred_element_type=jnp.float32
    )

    @pl.when(m == 0)
    def _init():
        out_ref[...] = z

    @pl.when(m != 0)
    def _accum():
        out_ref[...] += z


def _tc_gather(hist_t, off_tc, t_tc):
    grid_spec = pltpu.PrefetchScalarGridSpec(
        num_scalar_prefetch=0,
        grid=(_TC_STEPS,),
        in_specs=[
            pl.BlockSpec(memory_space=pltpu.SMEM),
            pl.BlockSpec((1, 1, _TC_GROUP), lambda g: (g, 0, 0)),
            pl.BlockSpec(memory_space=pl.ANY),
        ],
        out_specs=pl.BlockSpec(
            (_NUM_CLASSES, 128), lambda g: (0, g // 16)
        ),
        scratch_shapes=[
            pltpu.VMEM((2, _NUM_CLASSES, 128 * _TC_GROUP), jnp.float32),
            pltpu.SemaphoreType.DMA((2,)),
        ],
    )
    return pl.pallas_call(
        _tc_gather_body,
        grid_spec=grid_spec,
        out_shape=jax.ShapeDtypeStruct((_NUM_CLASSES, _NTC), jnp.float32),
    )(off_tc, t_tc, hist_t)


def _tc_softmax_body(out_ref, tgt_ref, y_ref, ce_ref):
    x = out_ref[...]
    m = jnp.max(x, axis=0, keepdims=True)
    xm = x - m
    e = jnp.exp(xm)
    s = jnp.sum(e, axis=0, keepdims=True)
    y_ref[...] = jnp.clip(e / s, 0.0001, 1.0 - 0.0001)
    log_sm = xm - jnp.log(s)
    ce_ref[0, 0] = jnp.sum(-tgt_ref[...] * log_sm)


def _tc_softmax(out_t, tgt_t):
    return pl.pallas_call(
        _tc_softmax_body,
        out_shape=(
            jax.ShapeDtypeStruct((_NUM_CLASSES, _BATCH), jnp.float32),
            jax.ShapeDtypeStruct((1, 1), jnp.float32),
        ),
        in_specs=[
            pl.BlockSpec(memory_space=pltpu.VMEM),
            pl.BlockSpec(memory_space=pltpu.VMEM),
        ],
        out_specs=(
            pl.BlockSpec(memory_space=pltpu.VMEM),
            pl.BlockSpec(memory_space=pltpu.SMEM),
        ),
    )(out_t, tgt_t)


def _tc_final_body(hsc_ref, htc_ref, y_ref, ce_ref, loss_ref):
    ii = lax.broadcasted_iota(jnp.int32, (_NUM_CLASSES, _NUM_CLASSES), 0)
    jj = lax.broadcasted_iota(jnp.int32, (_NUM_CLASSES, _NUM_CLASSES), 1)
    eye = jnp.where(ii == jj, 1.0, 0.0).astype(jnp.float32)
    hsc_t = lax.dot_general(
        eye,
        hsc_ref[...],
        (((1,), (1,)), ((), ())),
        preferred_element_type=jnp.float32,
    )
    y = y_ref[...]
    dot_a = jnp.sum(hsc_t * y[:, : _F], axis=0, keepdims=True)
    dot_b = jnp.sum(htc_ref[...] * y[:, _F:], axis=0, keepdims=True)
    reg = jnp.sum(jnp.log(1.0 - dot_a)) + jnp.sum(jnp.log(1.0 - dot_b))
    loss_ref[0, 0] = (ce_ref[0, 0] + _LAMBDA * reg) / _BATCH


def _tc_final(hist_sc, hist_tc_t, y_t, ce):
    return pl.pallas_call(
        _tc_final_body,
        out_shape=jax.ShapeDtypeStruct((1, 1), jnp.float32),
        in_specs=[
            pl.BlockSpec(memory_space=pltpu.VMEM),
            pl.BlockSpec(memory_space=pltpu.VMEM),
            pl.BlockSpec(memory_space=pltpu.VMEM),
            pl.BlockSpec(memory_space=pltpu.SMEM),
        ],
        out_specs=pl.BlockSpec(memory_space=pltpu.SMEM),
    )(hist_sc, hist_tc_t, y_t, ce)


def kernel(index, output, target, history):
    idx = index.astype(jnp.int32)
    t_all = jnp.bitwise_and(idx, 127)
    off_all = idx - t_all
    hist_t = history.T
    hist_sc = _sc_gather(hist_t, idx[:_F])
    hist_tc_t = _tc_gather(
        hist_t,
        off_all[_F:],
        jnp.reshape(t_all[_F:], (_TC_STEPS, 1, _TC_GROUP)),
    )
    y_t, ce = _tc_softmax(output.T, target.T)
    loss = _tc_final(hist_sc, hist_tc_t, y_t, ce)
    return loss[0, 0]


# speedup vs baseline: 2.1718x; 1.0026x over previous
import functools

import jax
import jax.numpy as jnp
from jax import lax
from jax.experimental import pallas as pl
from jax.experimental.pallas import tpu as pltpu
from jax.experimental.pallas import tpu_sc as plsc

_NUM_CLASSES = 100
_BATCH = 4096
_LAMBDA = 3.0
_NUM_WORKERS = 32
_F = 3584
_NTC = _BATCH - _F
_B_PER_W = _F // _NUM_WORKERS
_NSLAB = 8
_TC_GROUP = 8
_TC_STEPS = _NTC // _TC_GROUP


def _sc_gather(hist_t, idx):
    mesh = plsc.VectorSubcoreMesh(core_axis_name="c", subcore_axis_name="s")

    @functools.partial(
        pl.kernel,
        out_type=jax.ShapeDtypeStruct((_F, _NUM_CLASSES), jnp.float32),
        mesh=mesh,
        scratch_types=[
            pltpu.VMEM((_B_PER_W,), jnp.int32),
            [pltpu.VMEM((_NUM_CLASSES, 128), jnp.float32)] * _NSLAB,
            pltpu.VMEM((_B_PER_W, _NUM_CLASSES), jnp.float32),
            pltpu.SemaphoreType.DMA,
            pltpu.SemaphoreType.DMA,
        ],
        compiler_params=pltpu.CompilerParams(needs_layout_passes=False),
    )
    def gather_kernel(
        hist_hbm, idx_hbm, out_hbm, idx_v, slabs_v, compact_v, sem_a, sem_b
    ):
        wid = lax.axis_index("s") * 2 + lax.axis_index("c")
        base = wid * _B_PER_W
        pltpu.sync_copy(idx_hbm.at[pl.ds(base, _B_PER_W)], idx_v)

        iota16 = lax.iota(jnp.int32, 16)
        chunk_starts = [0, 16, 32, 48, 64, 80, 84]
        sems = [sem_a, sem_b]

        def issue(v, g, sem):
            ts = []
            for u in range(4):
                r = v[4 * g + u]
                t = lax.bitwise_and(r, 127)
                off = pl.multiple_of(lax.sub(r, t), 128)
                ts.append(t)
                pltpu.async_copy(
                    hist_hbm.at[:, pl.ds(off, 128)],
                    slabs_v[4 * (g % 2) + u],
                    sem,
                )
            return ts

        def extract(k, g, ts):
            for u in range(4):
                row = k * 16 + 4 * g + u
                tvec = iota16 * 0 + ts[u]
                for c0 in chunk_starts:
                    vals = plsc.load_gather(
                        slabs_v[4 * (g % 2) + u], [c0 + iota16, tvec]
                    )
                    compact_v[row, pl.ds(c0, 16)] = vals

        def drain(sem):
            for u in range(4):
                pltpu.make_async_copy(
                    hist_hbm.at[:, pl.ds(0, 128)], slabs_v[u], sem
                ).wait()

        def body(k, _):
            v = idx_v[pl.ds(k * 16, 16)]
            ts0 = issue(v, 0, sems[0])
            ts1 = issue(v, 1, sems[1])
            drain(sems[0])
            extract(k, 0, ts0)
            ts2 = issue(v, 2, sems[0])
            drain(sems[1])
            extract(k, 1, ts1)
            ts3 = issue(v, 3, sems[1])
            drain(sems[0])
            extract(k, 2, ts2)
            drain(sems[1])
            extract(k, 3, ts3)
            return 0

        lax.fori_loop(0, _B_PER_W // 16, body, 0)
        pltpu.sync_copy(compact_v, out_hbm.at[pl.ds(base, _B_PER_W)])

    return gather_kernel(hist_t, idx)


def _tc_gather_body(off_ref, t_ref, hist_ref, out_ref, slabs_ref, sems):
    g = pl.program_id(0)

    def issue(gg, b):
        for u in range(_TC_GROUP):
            r = off_ref[_F + gg * _TC_GROUP + u]
            off = pl.multiple_of(r - lax.bitwise_and(r, 127), 128)
            pltpu.make_async_copy(
                hist_ref.at[:, pl.ds(off, 128)],
                slabs_ref.at[b, :, pl.ds(128 * u, 128)],
                sems.at[b],
            ).start()

    @pl.when(g == 0)
    def _prologue():
        issue(0, 0)

    @pl.when(g + 1 < _TC_STEPS)
    def _next():
        issue(g + 1, (g + 1) % 2)

    b = g % 2
    pltpu.make_async_copy(
        hist_ref.at[:, pl.ds(0, 128 * _TC_GROUP)],
        slabs_ref.at[b],
        sems.at[b],
    ).wait()

    s = slabs_ref[b]
    tcol = jnp.reshape(
        jnp.bitwise_and(t_ref[0, 0, :], 127), (_TC_GROUP, 1)
    )
    li = lax.broadcasted_iota(jnp.int32, (_TC_GROUP, 128 * _TC_GROUP), 1)
    si = lax.broadcasted_iota(jnp.int32, (_TC_GROUP, 128 * _TC_GROUP), 0)
    p8 = jnp.where(
        (lax.shift_right_logical(li, 7) == si)
        & (lax.bitwise_and(li, 127) == tcol),
        1.0,
        0.0,
    ).astype(jnp.float32)
    e = lax.dot_general(
        s, p8, (((1,), (1,)), ((), ())), preferred_element_type=jnp.float32
    )
    m = g % 16
    ri = lax.broadcasted_iota(jnp.int32, (_TC_GROUP, 128), 0)
    rl = lax.broadcasted_iota(jnp.int32, (_TC_GROUP, 128), 1)
    rmat = jnp.where(rl == m * _TC_GROUP + ri, 1.0, 0.0).astype(jnp.float32)
    z = lax.dot_general(
        e, rmat, (((1,), (0,)), ((), ())), preferred_element_type=jnp.float32
    )

    @pl.when(m == 0)
    def _init():
        out_ref[...] = z

    @pl.when(m != 0)
    def _accum():
        out_ref[...] += z


def _tc_gather(hist_t, idx, idx3):
    grid_spec = pltpu.PrefetchScalarGridSpec(
        num_scalar_prefetch=0,
        grid=(_TC_STEPS,),
        in_specs=[
            pl.BlockSpec(memory_space=pltpu.SMEM),
            pl.BlockSpec(
                (1, 1, _TC_GROUP),
                lambda g: (g + _F // _TC_GROUP, 0, 0),
            ),
            pl.BlockSpec(memory_space=pl.ANY),
        ],
        out_specs=pl.BlockSpec(
            (_NUM_CLASSES, 128), lambda g: (0, g // 16)
        ),
        scratch_shapes=[
            pltpu.VMEM((2, _NUM_CLASSES, 128 * _TC_GROUP), jnp.float32),
            pltpu.SemaphoreType.DMA((2,)),
        ],
    )
    return pl.pallas_call(
        _tc_gather_body,
        grid_spec=grid_spec,
        out_shape=jax.ShapeDtypeStruct((_NUM_CLASSES, _NTC), jnp.float32),
    )(idx, idx3, hist_t)


def _tc_softmax_body(out_ref, tgt_ref, y_ref, ce_ref):
    x = out_ref[...]
    m = jnp.max(x, axis=0, keepdims=True)
    xm = x - m
    e = jnp.exp(xm)
    s = jnp.sum(e, axis=0, keepdims=True)
    y_ref[...] = jnp.clip(e / s, 0.0001, 1.0 - 0.0001)
    log_sm = xm - jnp.log(s)
    ce_ref[0, 0] = jnp.sum(-tgt_ref[...] * log_sm)


def _tc_softmax(out_t, tgt_t):
    return pl.pallas_call(
        _tc_softmax_body,
        out_shape=(
            jax.ShapeDtypeStruct((_NUM_CLASSES, _BATCH), jnp.float32),
            jax.ShapeDtypeStruct((1, 1), jnp.float32),
        ),
        in_specs=[
            pl.BlockSpec(memory_space=pltpu.VMEM),
            pl.BlockSpec(memory_space=pltpu.VMEM),
        ],
        out_specs=(
            pl.BlockSpec(memory_space=pltpu.VMEM),
            pl.BlockSpec(memory_space=pltpu.SMEM),
        ),
    )(out_t, tgt_t)


def _tc_final_body(hsc_ref, htc_ref, y_ref, ce_ref, loss_ref):
    ii = lax.broadcasted_iota(jnp.int32, (_NUM_CLASSES, _NUM_CLASSES), 0)
    jj = lax.broadcasted_iota(jnp.int32, (_NUM_CLASSES, _NUM_CLASSES), 1)
    eye = jnp.where(ii == jj, 1.0, 0.0).astype(jnp.float32)
    hsc_t = lax.dot_general(
        eye,
        hsc_ref[...],
        (((1,), (1,)), ((), ())),
        preferred_element_type=jnp.float32,
    )
    y = y_ref[...]
    dot_a = jnp.sum(hsc_t * y[:, : _F], axis=0, keepdims=True)
    dot_b = jnp.sum(htc_ref[...] * y[:, _F:], axis=0, keepdims=True)
    reg = jnp.sum(jnp.log(1.0 - dot_a)) + jnp.sum(jnp.log(1.0 - dot_b))
    loss_ref[0, 0] = (ce_ref[0, 0] + _LAMBDA * reg) / _BATCH


def _tc_final(hist_sc, hist_tc_t, y_t, ce):
    return pl.pallas_call(
        _tc_final_body,
        out_shape=jax.ShapeDtypeStruct((1, 1), jnp.float32),
        in_specs=[
            pl.BlockSpec(memory_space=pltpu.VMEM),
            pl.BlockSpec(memory_space=pltpu.VMEM),
            pl.BlockSpec(memory_space=pltpu.VMEM),
            pl.BlockSpec(memory_space=pltpu.SMEM),
        ],
        out_specs=pl.BlockSpec(memory_space=pltpu.SMEM),
    )(hist_sc, hist_tc_t, y_t, ce)


def kernel(index, output, target, history):
    idx = index.astype(jnp.int32)
    hist_t = history.T
    hist_sc = _sc_gather(hist_t, idx)
    hist_tc_t = _tc_gather(
        hist_t, idx, jnp.reshape(idx, (_BATCH // _TC_GROUP, 1, _TC_GROUP))
    )
    y_t, ce = _tc_softmax(output.T, target.T)
    loss = _tc_final(hist_sc, hist_tc_t, y_t, ce)
    return loss[0, 0]
